# baseline (speedup 1.0000x reference)
import functools
import math

import jax
import jax.numpy as jnp
from jax import lax
from jax.experimental import pallas as pl
from jax.experimental.pallas import tpu as pltpu
from jax.experimental.pallas import tpu_sc as plsc

D_M = 64
SCALE = math.sqrt(D_M)
LANES = 16
CHUNK = 128


@functools.lru_cache(maxsize=None)
def _build(n_ch: int, num_cores: int, num_subcores: int):
    nw = num_cores * num_subcores
    mesh = plsc.VectorSubcoreMesh(core_axis_name="c", subcore_axis_name="s")
    b_total = nw * n_ch * CHUNK

    @functools.partial(
        pl.kernel,
        mesh=mesh,
        out_type=jax.ShapeDtypeStruct((b_total, D_M), jnp.float32),
        compiler_params=pltpu.CompilerParams(use_tc_tiling_on_sc=False),
        scratch_types=[
            pltpu.VMEM((n_ch, CHUNK), jnp.int32),
            pltpu.VMEM((CHUNK, D_M), jnp.float32),
            pltpu.SemaphoreType.DMA,
        ],
    )
    def emb(x_hbm, tab_hbm, out_hbm, idx_v, rows_v, sem):
        wid = lax.axis_index("s") * num_cores + lax.axis_index("c")
        pltpu.sync_copy(x_hbm.at[pl.ds(wid * n_ch, n_ch)], idx_v)

        def chunk_body(j, carry):
            pltpu.async_copy(tab_hbm.at[idx_v.at[j]], rows_v, sem).wait()

            def scale_row(r, c2):
                for c in range(D_M // LANES):
                    sl = pl.ds(c * LANES, LANES)
                    rows_v[r, sl] = rows_v[r, sl] * SCALE
                return c2

            lax.fori_loop(0, CHUNK, scale_row, 0)
            pltpu.sync_copy(
                rows_v, out_hbm.at[pl.ds((wid * n_ch + j) * CHUNK, CHUNK)]
            )
            return carry

        lax.fori_loop(0, n_ch, chunk_body, 0)

    return emb


def kernel(x, table):
    rows, cols = x.shape
    b_total = rows * cols
    info = plsc.get_sparse_core_info()
    nw = info.num_cores * info.num_subcores
    assert b_total % (nw * CHUNK) == 0
    n_ch = b_total // (nw * CHUNK)
    xf = x.reshape(nw * n_ch, CHUNK).astype(jnp.int32)
    emb = _build(n_ch, info.num_cores, info.num_subcores)
    out = emb(xf, table)
    return out.reshape(rows, cols, D_M)


# speedup vs baseline: 1.2089x; 1.2089x over previous
import functools
import math

import jax
import jax.numpy as jnp
from jax import lax
from jax.experimental import pallas as pl
from jax.experimental.pallas import tpu as pltpu
from jax.experimental.pallas import tpu_sc as plsc

D_M = 64
SCALE = math.sqrt(D_M)
LANES = 16
CHUNK = 128
NBUF = 4


@functools.lru_cache(maxsize=None)
def _build(n_ch: int, num_cores: int, num_subcores: int):
    nw = num_cores * num_subcores
    mesh = plsc.VectorSubcoreMesh(core_axis_name="c", subcore_axis_name="s")
    b_total = nw * n_ch * CHUNK
    assert n_ch % NBUF == 0 and n_ch >= 2 * NBUF

    @functools.partial(
        pl.kernel,
        mesh=mesh,
        out_type=jax.ShapeDtypeStruct((b_total, D_M), jnp.float32),
        compiler_params=pltpu.CompilerParams(use_tc_tiling_on_sc=False),
        scratch_types=[
            pltpu.VMEM((n_ch, CHUNK), jnp.int32),
            pltpu.VMEM((NBUF, CHUNK, D_M), jnp.float32),
            pltpu.VMEM((NBUF, CHUNK, D_M), jnp.float32),
            [pltpu.SemaphoreType.DMA] * NBUF,
            [pltpu.SemaphoreType.DMA] * NBUF,
        ],
    )
    def emb(x_hbm, tab_hbm, out_hbm, idx_v, gbuf, sbuf, gsems, ssems):
        wid = lax.axis_index("s") * num_cores + lax.axis_index("c")
        base = wid * n_ch
        pltpu.sync_copy(x_hbm.at[pl.ds(base, n_ch)], idx_v)

        for b in range(NBUF):
            pltpu.async_copy(tab_hbm.at[idx_v.at[b]], gbuf.at[b], gsems[b])

        def scale(b):
            def row(r, c2):
                for c in range(D_M // LANES):
                    sl = pl.ds(c * LANES, LANES)
                    sbuf[b, r, sl] = gbuf[b, r, sl] * SCALE
                return c2

            lax.fori_loop(0, CHUNK, row, 0)

        @pl.loop(0, n_ch, step=NBUF)
        def outer(j0):
            for b in range(NBUF):
                k = j0 + b
                pltpu.make_async_copy(
                    tab_hbm.at[pl.ds(0, CHUNK)], gbuf.at[b], gsems[b]
                ).wait()
                @pl.when(j0 > 0)
                def _():
                    pltpu.make_async_copy(
                        sbuf.at[b], out_hbm.at[pl.ds(0, CHUNK)], ssems[b]
                    ).wait()

                scale(b)
                @pl.when(k + NBUF < n_ch)
                def _():
                    pltpu.async_copy(
                        tab_hbm.at[idx_v.at[k + NBUF]], gbuf.at[b], gsems[b]
                    )

                pltpu.async_copy(
                    sbuf.at[b],
                    out_hbm.at[pl.ds((base + k) * CHUNK, CHUNK)],
                    ssems[b],
                )

        for b in range(NBUF):
            pltpu.make_async_copy(
                sbuf.at[b], out_hbm.at[pl.ds(0, CHUNK)], ssems[b]
            ).wait()

    return emb


def kernel(x, table):
    rows, cols = x.shape
    b_total = rows * cols
    info = plsc.get_sparse_core_info()
    nw = info.num_cores * info.num_subcores
    assert b_total % (nw * CHUNK) == 0
    n_ch = b_total // (nw * CHUNK)
    xf = x.reshape(nw * n_ch, CHUNK).astype(jnp.int32)
    emb = _build(n_ch, info.num_cores, info.num_subcores)
    out = emb(xf, table)
    return out.reshape(rows, cols, D_M)
